# baseline (speedup 1.0000x reference)
import functools

import jax
import jax.numpy as jnp
from jax import lax
from jax.experimental import pallas as pl
from jax.experimental.pallas import tpu as pltpu
from jax.experimental.pallas import tpu_sc as plsc

B = 16384
D = 32
H = 64
NROWS = 1000000
RPP = 128 // D
NC = 2
NS = 16
NW = NC * NS
BPW = B // NW
CHUNK = 128
NCHUNK = BPW // CHUNK


_DBLK = 1000


def _depad_body(ut_ref, it_ref, uo_ref, io_ref):
    uo_ref[...] = ut_ref[...].reshape(_DBLK, 256)
    io_ref[...] = it_ref[...].reshape(_DBLK, 256)


_depad = pl.pallas_call(
    _depad_body,
    grid=(NROWS // 8 // _DBLK,),
    in_specs=[
        pl.BlockSpec((_DBLK, 8, D), lambda i: (i, 0, 0)),
        pl.BlockSpec((_DBLK, 8, D), lambda i: (i, 0, 0)),
    ],
    out_specs=[
        pl.BlockSpec((_DBLK, 256), lambda i: (i, 0)),
        pl.BlockSpec((_DBLK, 256), lambda i: (i, 0)),
    ],
    out_shape=[
        jax.ShapeDtypeStruct((NROWS // 8, 256), jnp.float32),
        jax.ShapeDtypeStruct((NROWS // 8, 256), jnp.float32),
    ],
)


_mesh = plsc.VectorSubcoreMesh(core_axis_name="c", subcore_axis_name="s")


@functools.partial(
    pl.kernel,
    mesh=_mesh,
    out_type=[
        jax.ShapeDtypeStruct((B, 128), jnp.float32),
        jax.ShapeDtypeStruct((B, 128), jnp.float32),
    ],
    scratch_types=[
        pltpu.VMEM((NCHUNK, CHUNK), jnp.int32),
        pltpu.VMEM((NCHUNK, CHUNK), jnp.int32),
        pltpu.VMEM((BPW // 2, 128), jnp.float32),
        pltpu.VMEM((BPW // 2, 128), jnp.float32),
        pltpu.SemaphoreType.DMA,
        pltpu.SemaphoreType.DMA,
    ],
)
def _gather_uv(uid_hbm, iid_hbm, ut_hbm, it_hbm, u_out, v_out,
               uidx, iidx, urows, vrows, usem, vsem):
    wid = lax.axis_index("s") * NC + lax.axis_index("c")
    base = wid * BPW
    pltpu.sync_copy(uid_hbm.at[pl.ds(wid * NCHUNK, NCHUNK)], uidx)
    pltpu.sync_copy(iid_hbm.at[pl.ds(wid * NCHUNK, NCHUNK)], iidx)
    for half in range(2):
        cps = []
        for j in range(NCHUNK // 2):
            jj = half * (NCHUNK // 2) + j
            cps.append(pltpu.async_copy(
                ut_hbm.at[uidx.at[jj]],
                urows.at[pl.ds(j * CHUNK, CHUNK)], usem))
            cps.append(pltpu.async_copy(
                it_hbm.at[iidx.at[jj]],
                vrows.at[pl.ds(j * CHUNK, CHUNK)], vsem))
        for cp in cps:
            cp.wait()
        pltpu.sync_copy(urows, u_out.at[pl.ds(base + half * (BPW // 2),
                                              BPW // 2)])
        pltpu.sync_copy(vrows, v_out.at[pl.ds(base + half * (BPW // 2),
                                              BPW // 2)])


_BLK = 2048


def _mlp_body(u_ref, v_ref, ru_ref, rv_ref, w1u_ref, w1v_ref, b1_ref,
              w2_ref, b2_ref, o_ref):
    u128 = u_ref[...]
    v128 = v_ref[...]
    ru = ru_ref[...]
    rv = rv_ref[...]
    u = jnp.where(ru == 0., u128[:, 0:D], 0.)
    v = jnp.where(rv == 0., v128[:, 0:D], 0.)
    for k in range(1, RPP):
        u = u + jnp.where(ru == float(k), u128[:, k * D:(k + 1) * D], 0.)
        v = v + jnp.where(rv == float(k), v128[:, k * D:(k + 1) * D], 0.)
    h = jnp.dot(u, w1u_ref[...], preferred_element_type=jnp.float32)
    h = h + jnp.dot(v, w1v_ref[...], preferred_element_type=jnp.float32)
    h = jnp.maximum(h + b1_ref[...], 0.0)
    y = jnp.sum(h * w2_ref[...], axis=1)
    o_ref[...] = (y[None, :] + b2_ref[...])[None]


_mlp = pl.pallas_call(
    _mlp_body,
    grid=(B // _BLK,),
    in_specs=[
        pl.BlockSpec((_BLK, 128), lambda i: (i, 0)),
        pl.BlockSpec((_BLK, 128), lambda i: (i, 0)),
        pl.BlockSpec((_BLK, 1), lambda i: (i, 0)),
        pl.BlockSpec((_BLK, 1), lambda i: (i, 0)),
        pl.BlockSpec((D, H), lambda i: (0, 0)),
        pl.BlockSpec((D, H), lambda i: (0, 0)),
        pl.BlockSpec((1, H), lambda i: (0, 0)),
        pl.BlockSpec((1, H), lambda i: (0, 0)),
        pl.BlockSpec((1, 1), lambda i: (0, 0)),
    ],
    out_specs=pl.BlockSpec((1, 1, _BLK), lambda i: (i, 0, 0)),
    out_shape=jax.ShapeDtypeStruct((B // _BLK, 1, _BLK), jnp.float32),
)


def kernel(userID, ItemID, user_table, item_table, W1, b1, W2, b2):
    uid = userID.astype(jnp.int32)
    iid = ItemID.astype(jnp.int32)
    upix = (uid // RPP).reshape(B // CHUNK, CHUNK)
    ipix = (iid // RPP).reshape(B // CHUNK, CHUNK)
    ut_c, it_c = _depad(user_table.reshape(-1, 8, D),
                        item_table.reshape(-1, 8, D))
    ut_c = ut_c.reshape(-1, 128)
    it_c = it_c.reshape(-1, 128)
    u128, v128 = _gather_uv(upix, ipix, ut_c, it_c)
    ru = (uid % RPP).astype(jnp.float32).reshape(B, 1)
    rv = (iid % RPP).astype(jnp.float32).reshape(B, 1)
    y = _mlp(u128, v128, ru, rv, W1[:D], W1[D:], b1.reshape(1, H),
             W2.reshape(1, H), b2.reshape(1, 1))
    return y.reshape(B)


# speedup vs baseline: 1.1577x; 1.1577x over previous
import functools

import jax
import jax.numpy as jnp
from jax import lax
from jax.experimental import pallas as pl
from jax.experimental.pallas import tpu as pltpu
from jax.experimental.pallas import tpu_sc as plsc

B = 16384
D = 32
H = 64
NROWS = 1000000
RPP = 128 // D
NC = 2
NS = 16
NW = NC * NS
BPW = B // NW
CHUNK = 128
NCHUNK = BPW // CHUNK


_TBLK = 1024
_TGRID = -(-NROWS // _TBLK)


def _transp_body(ut_ref, it_ref, uo_ref, io_ref):
    x = ut_ref[...]
    z = it_ref[...]
    uo_ref[...] = jnp.concatenate(
        [jnp.transpose(x[:, a * 256:(a + 1) * 256]) for a in range(RPP)],
        axis=1)
    io_ref[...] = jnp.concatenate(
        [jnp.transpose(z[:, a * 256:(a + 1) * 256]) for a in range(RPP)],
        axis=1)


_transp = pl.pallas_call(
    _transp_body,
    grid=(_TGRID,),
    in_specs=[
        pl.BlockSpec((D, _TBLK), lambda i: (0, i)),
        pl.BlockSpec((D, _TBLK), lambda i: (0, i)),
    ],
    out_specs=[
        pl.BlockSpec((256, 128), lambda i: (i, 0)),
        pl.BlockSpec((256, 128), lambda i: (i, 0)),
    ],
    out_shape=[
        jax.ShapeDtypeStruct((_TGRID * 256, 128), jnp.float32),
        jax.ShapeDtypeStruct((_TGRID * 256, 128), jnp.float32),
    ],
)


_mesh = plsc.VectorSubcoreMesh(core_axis_name="c", subcore_axis_name="s")


@functools.partial(
    pl.kernel,
    mesh=_mesh,
    out_type=[
        jax.ShapeDtypeStruct((B, 128), jnp.float32),
        jax.ShapeDtypeStruct((B, 128), jnp.float32),
    ],
    scratch_types=[
        pltpu.VMEM((NCHUNK, CHUNK), jnp.int32),
        pltpu.VMEM((NCHUNK, CHUNK), jnp.int32),
        pltpu.VMEM((BPW // 2, 128), jnp.float32),
        pltpu.VMEM((BPW // 2, 128), jnp.float32),
        pltpu.SemaphoreType.DMA,
        pltpu.SemaphoreType.DMA,
    ],
)
def _gather_uv(uid_hbm, iid_hbm, ut_hbm, it_hbm, u_out, v_out,
               uidx, iidx, urows, vrows, usem, vsem):
    wid = lax.axis_index("s") * NC + lax.axis_index("c")
    base = wid * BPW
    pltpu.sync_copy(uid_hbm.at[pl.ds(wid * NCHUNK, NCHUNK)], uidx)
    pltpu.sync_copy(iid_hbm.at[pl.ds(wid * NCHUNK, NCHUNK)], iidx)
    for half in range(2):
        cps = []
        for j in range(NCHUNK // 2):
            jj = half * (NCHUNK // 2) + j
            cps.append(pltpu.async_copy(
                ut_hbm.at[uidx.at[jj]],
                urows.at[pl.ds(j * CHUNK, CHUNK)], usem))
            cps.append(pltpu.async_copy(
                it_hbm.at[iidx.at[jj]],
                vrows.at[pl.ds(j * CHUNK, CHUNK)], vsem))
        for cp in cps:
            cp.wait()
        pltpu.sync_copy(urows, u_out.at[pl.ds(base + half * (BPW // 2),
                                              BPW // 2)])
        pltpu.sync_copy(vrows, v_out.at[pl.ds(base + half * (BPW // 2),
                                              BPW // 2)])


_BLK = 2048


def _mlp_body(u_ref, v_ref, ru_ref, rv_ref, w1u_ref, w1v_ref, b1_ref,
              w2_ref, b2_ref, o_ref):
    u128 = u_ref[...]
    v128 = v_ref[...]
    ru = ru_ref[...]
    rv = rv_ref[...]
    u = jnp.where(ru == 0., u128[:, 0:D], 0.)
    v = jnp.where(rv == 0., v128[:, 0:D], 0.)
    for k in range(1, RPP):
        u = u + jnp.where(ru == float(k), u128[:, k * D:(k + 1) * D], 0.)
        v = v + jnp.where(rv == float(k), v128[:, k * D:(k + 1) * D], 0.)
    h = jnp.dot(u, w1u_ref[...], preferred_element_type=jnp.float32)
    h = h + jnp.dot(v, w1v_ref[...], preferred_element_type=jnp.float32)
    h = jnp.maximum(h + b1_ref[...], 0.0)
    y = jnp.sum(h * w2_ref[...], axis=1)
    o_ref[...] = (y[None, :] + b2_ref[...])[None]


_mlp = pl.pallas_call(
    _mlp_body,
    grid=(B // _BLK,),
    in_specs=[
        pl.BlockSpec((_BLK, 128), lambda i: (i, 0)),
        pl.BlockSpec((_BLK, 128), lambda i: (i, 0)),
        pl.BlockSpec((_BLK, 1), lambda i: (i, 0)),
        pl.BlockSpec((_BLK, 1), lambda i: (i, 0)),
        pl.BlockSpec((D, H), lambda i: (0, 0)),
        pl.BlockSpec((D, H), lambda i: (0, 0)),
        pl.BlockSpec((1, H), lambda i: (0, 0)),
        pl.BlockSpec((1, H), lambda i: (0, 0)),
        pl.BlockSpec((1, 1), lambda i: (0, 0)),
    ],
    out_specs=pl.BlockSpec((1, 1, _BLK), lambda i: (i, 0, 0)),
    out_shape=jax.ShapeDtypeStruct((B // _BLK, 1, _BLK), jnp.float32),
)


def kernel(userID, ItemID, user_table, item_table, W1, b1, W2, b2):
    uid = userID.astype(jnp.int32)
    iid = ItemID.astype(jnp.int32)
    upix = ((uid // _TBLK) * 256 + uid % 256).reshape(B // CHUNK, CHUNK)
    ipix = ((iid // _TBLK) * 256 + iid % 256).reshape(B // CHUNK, CHUNK)
    ut_c, it_c = _transp(user_table.T, item_table.T)
    u128, v128 = _gather_uv(upix, ipix, ut_c, it_c)
    ru = ((uid // 256) % RPP).astype(jnp.float32).reshape(B, 1)
    rv = ((iid // 256) % RPP).astype(jnp.float32).reshape(B, 1)
    y = _mlp(u128, v128, ru, rv, W1[:D], W1[D:], b1.reshape(1, H),
             W2.reshape(1, H), b2.reshape(1, 1))
    return y.reshape(B)


# speedup vs baseline: 1.2213x; 1.0550x over previous
import functools

import jax
import jax.numpy as jnp
from jax import lax
from jax.experimental import pallas as pl
from jax.experimental.pallas import tpu as pltpu
from jax.experimental.pallas import tpu_sc as plsc

B = 16384
D = 32
H = 64
NROWS = 1000000
RPP = 128 // D
NC = 2
NS = 16
NW = NC * NS
BPW = B // NW
CHUNK = 128
NCHUNK = BPW // CHUNK


_TBLK = 1024
_TGRID = -(-NROWS // _TBLK)


def _transp_body(ut_ref, it_ref, uo_ref, io_ref):
    x = ut_ref[...]
    z = it_ref[...]
    row = jax.lax.broadcasted_iota(jnp.int32, (D, 128), 0)
    col = jax.lax.broadcasted_iota(jnp.int32, (D, 128), 1)
    u = jnp.zeros((256, 128), jnp.float32)
    v = jnp.zeros((256, 128), jnp.float32)
    for a in range(RPP):
        ea = (col == row + a * D).astype(jnp.float32)
        dn = (((0,), (0,)), ((), ()))
        u = u + jax.lax.dot_general(x[:, a * 256:(a + 1) * 256], ea, dn,
                                    preferred_element_type=jnp.float32)
        v = v + jax.lax.dot_general(z[:, a * 256:(a + 1) * 256], ea, dn,
                                    preferred_element_type=jnp.float32)
    uo_ref[...] = u
    io_ref[...] = v


_transp = pl.pallas_call(
    _transp_body,
    grid=(_TGRID,),
    in_specs=[
        pl.BlockSpec((D, _TBLK), lambda i: (0, i)),
        pl.BlockSpec((D, _TBLK), lambda i: (0, i)),
    ],
    out_specs=[
        pl.BlockSpec((256, 128), lambda i: (i, 0)),
        pl.BlockSpec((256, 128), lambda i: (i, 0)),
    ],
    out_shape=[
        jax.ShapeDtypeStruct((_TGRID * 256, 128), jnp.float32),
        jax.ShapeDtypeStruct((_TGRID * 256, 128), jnp.float32),
    ],
    compiler_params=pltpu.CompilerParams(fuse_transposed_lhs_in_matmul=True),
)


_mesh = plsc.VectorSubcoreMesh(core_axis_name="c", subcore_axis_name="s")


@functools.partial(
    pl.kernel,
    mesh=_mesh,
    out_type=[
        jax.ShapeDtypeStruct((B, 128), jnp.float32),
        jax.ShapeDtypeStruct((B, 128), jnp.float32),
    ],
    scratch_types=[
        pltpu.VMEM((NCHUNK, CHUNK), jnp.int32),
        pltpu.VMEM((NCHUNK, CHUNK), jnp.int32),
        pltpu.VMEM((BPW // 2, 128), jnp.float32),
        pltpu.VMEM((BPW // 2, 128), jnp.float32),
        pltpu.SemaphoreType.DMA,
        pltpu.SemaphoreType.DMA,
    ],
)
def _gather_uv(uid_hbm, iid_hbm, ut_hbm, it_hbm, u_out, v_out,
               uidx, iidx, urows, vrows, usem, vsem):
    wid = lax.axis_index("s") * NC + lax.axis_index("c")
    base = wid * BPW
    pltpu.sync_copy(uid_hbm.at[pl.ds(wid * NCHUNK, NCHUNK)], uidx)
    pltpu.sync_copy(iid_hbm.at[pl.ds(wid * NCHUNK, NCHUNK)], iidx)
    for half in range(2):
        cps = []
        for j in range(NCHUNK // 2):
            jj = half * (NCHUNK // 2) + j
            cps.append(pltpu.async_copy(
                ut_hbm.at[uidx.at[jj]],
                urows.at[pl.ds(j * CHUNK, CHUNK)], usem))
            cps.append(pltpu.async_copy(
                it_hbm.at[iidx.at[jj]],
                vrows.at[pl.ds(j * CHUNK, CHUNK)], vsem))
        for cp in cps:
            cp.wait()
        pltpu.sync_copy(urows, u_out.at[pl.ds(base + half * (BPW // 2),
                                              BPW // 2)])
        pltpu.sync_copy(vrows, v_out.at[pl.ds(base + half * (BPW // 2),
                                              BPW // 2)])


_BLK = 2048


def _mlp_body(u_ref, v_ref, ru_ref, rv_ref, w1u_ref, w1v_ref, b1_ref,
              w2_ref, b2_ref, o_ref):
    u128 = u_ref[...]
    v128 = v_ref[...]
    ru = ru_ref[...]
    rv = rv_ref[...]
    u = jnp.where(ru == 0., u128[:, 0:D], 0.)
    v = jnp.where(rv == 0., v128[:, 0:D], 0.)
    for k in range(1, RPP):
        u = u + jnp.where(ru == float(k), u128[:, k * D:(k + 1) * D], 0.)
        v = v + jnp.where(rv == float(k), v128[:, k * D:(k + 1) * D], 0.)
    h = jnp.dot(u, w1u_ref[...], preferred_element_type=jnp.float32)
    h = h + jnp.dot(v, w1v_ref[...], preferred_element_type=jnp.float32)
    h = jnp.maximum(h + b1_ref[...], 0.0)
    y = jnp.sum(h * w2_ref[...], axis=1)
    o_ref[...] = (y[None, :] + b2_ref[...])[None]


_mlp = pl.pallas_call(
    _mlp_body,
    grid=(B // _BLK,),
    in_specs=[
        pl.BlockSpec((_BLK, 128), lambda i: (i, 0)),
        pl.BlockSpec((_BLK, 128), lambda i: (i, 0)),
        pl.BlockSpec((_BLK, 1), lambda i: (i, 0)),
        pl.BlockSpec((_BLK, 1), lambda i: (i, 0)),
        pl.BlockSpec((D, H), lambda i: (0, 0)),
        pl.BlockSpec((D, H), lambda i: (0, 0)),
        pl.BlockSpec((1, H), lambda i: (0, 0)),
        pl.BlockSpec((1, H), lambda i: (0, 0)),
        pl.BlockSpec((1, 1), lambda i: (0, 0)),
    ],
    out_specs=pl.BlockSpec((1, 1, _BLK), lambda i: (i, 0, 0)),
    out_shape=jax.ShapeDtypeStruct((B // _BLK, 1, _BLK), jnp.float32),
)


def kernel(userID, ItemID, user_table, item_table, W1, b1, W2, b2):
    uid = userID.astype(jnp.int32)
    iid = ItemID.astype(jnp.int32)
    upix = ((uid // _TBLK) * 256 + uid % 256).reshape(B // CHUNK, CHUNK)
    ipix = ((iid // _TBLK) * 256 + iid % 256).reshape(B // CHUNK, CHUNK)
    ut_c, it_c = _transp(user_table.T, item_table.T)
    u128, v128 = _gather_uv(upix, ipix, ut_c, it_c)
    ru = ((uid // 256) % RPP).astype(jnp.float32).reshape(B, 1)
    rv = ((iid // 256) % RPP).astype(jnp.float32).reshape(B, 1)
    y = _mlp(u128, v128, ru, rv, W1[:D], W1[D:], b1.reshape(1, H),
             W2.reshape(1, H), b2.reshape(1, 1))
    return y.reshape(B)


# speedup vs baseline: 2.2749x; 1.8627x over previous
import functools

import jax
import jax.numpy as jnp
from jax import lax
from jax.experimental import pallas as pl
from jax.experimental.pallas import tpu as pltpu
from jax.experimental.pallas import tpu_sc as plsc

B = 16384
D = 32
H = 64
NROWS = 1000000
RPP = 128 // D
NC = 2
NS = 16
NW = NC * NS
BPW = B // NW
CHUNK = 128
NCHUNK = BPW // CHUNK


_TBLK = 4096
_TGRID = -(-NROWS // _TBLK)


def _transp_body(ut_ref, it_ref, uo_ref, io_ref):
    x = ut_ref[...]
    z = it_ref[...]
    row = jax.lax.broadcasted_iota(jnp.int32, (D, 128), 0)
    col = jax.lax.broadcasted_iota(jnp.int32, (D, 128), 1)
    dn = (((0,), (0,)), ((), ()))
    eas = [(col == row + a * D).astype(jnp.float32) for a in range(RPP)]
    for g in range(_TBLK // 1024):
        u = jnp.zeros((256, 128), jnp.float32)
        v = jnp.zeros((256, 128), jnp.float32)
        for a in range(RPP):
            c0 = g * 1024 + a * 256
            u = u + jax.lax.dot_general(x[:, c0:c0 + 256], eas[a], dn,
                                        preferred_element_type=jnp.float32)
            v = v + jax.lax.dot_general(z[:, c0:c0 + 256], eas[a], dn,
                                        preferred_element_type=jnp.float32)
        uo_ref[pl.ds(g * 256, 256), :] = u
        io_ref[pl.ds(g * 256, 256), :] = v


_transp = pl.pallas_call(
    _transp_body,
    grid=(_TGRID,),
    in_specs=[
        pl.BlockSpec((D, _TBLK), lambda i: (0, i)),
        pl.BlockSpec((D, _TBLK), lambda i: (0, i)),
    ],
    out_specs=[
        pl.BlockSpec((_TBLK // 4, 128), lambda i: (i, 0)),
        pl.BlockSpec((_TBLK // 4, 128), lambda i: (i, 0)),
    ],
    out_shape=[
        jax.ShapeDtypeStruct((_TGRID * (_TBLK // 4), 128), jnp.float32),
        jax.ShapeDtypeStruct((_TGRID * (_TBLK // 4), 128), jnp.float32),
    ],
    compiler_params=pltpu.CompilerParams(fuse_transposed_lhs_in_matmul=True),
)


_mesh = plsc.VectorSubcoreMesh(core_axis_name="c", subcore_axis_name="s")


@functools.partial(
    pl.kernel,
    mesh=_mesh,
    out_type=[
        jax.ShapeDtypeStruct((B, 128), jnp.float32),
        jax.ShapeDtypeStruct((B, 128), jnp.float32),
    ],
    scratch_types=[
        pltpu.VMEM((NCHUNK, CHUNK), jnp.int32),
        pltpu.VMEM((NCHUNK, CHUNK), jnp.int32),
        pltpu.VMEM((BPW // 2, 128), jnp.float32),
        pltpu.VMEM((BPW // 2, 128), jnp.float32),
        pltpu.SemaphoreType.DMA,
        pltpu.SemaphoreType.DMA,
    ],
)
def _gather_uv(uid_hbm, iid_hbm, ut_hbm, it_hbm, u_out, v_out,
               uidx, iidx, urows, vrows, usem, vsem):
    wid = lax.axis_index("s") * NC + lax.axis_index("c")
    base = wid * BPW
    pltpu.sync_copy(uid_hbm.at[pl.ds(wid * NCHUNK, NCHUNK)], uidx)
    pltpu.sync_copy(iid_hbm.at[pl.ds(wid * NCHUNK, NCHUNK)], iidx)
    for half in range(2):
        cps = []
        for j in range(NCHUNK // 2):
            jj = half * (NCHUNK // 2) + j
            cps.append(pltpu.async_copy(
                ut_hbm.at[uidx.at[jj]],
                urows.at[pl.ds(j * CHUNK, CHUNK)], usem))
            cps.append(pltpu.async_copy(
                it_hbm.at[iidx.at[jj]],
                vrows.at[pl.ds(j * CHUNK, CHUNK)], vsem))
        for cp in cps:
            cp.wait()
        pltpu.sync_copy(urows, u_out.at[pl.ds(base + half * (BPW // 2),
                                              BPW // 2)])
        pltpu.sync_copy(vrows, v_out.at[pl.ds(base + half * (BPW // 2),
                                              BPW // 2)])


_BLK = 2048


def _mlp_body(u_ref, v_ref, ru_ref, rv_ref, w1u_ref, w1v_ref, b1_ref,
              w2_ref, b2_ref, o_ref):
    u128 = u_ref[...]
    v128 = v_ref[...]
    ru = ru_ref[...]
    rv = rv_ref[...]
    u = jnp.where(ru == 0., u128[:, 0:D], 0.)
    v = jnp.where(rv == 0., v128[:, 0:D], 0.)
    for k in range(1, RPP):
        u = u + jnp.where(ru == float(k), u128[:, k * D:(k + 1) * D], 0.)
        v = v + jnp.where(rv == float(k), v128[:, k * D:(k + 1) * D], 0.)
    h = jnp.dot(u, w1u_ref[...], preferred_element_type=jnp.float32)
    h = h + jnp.dot(v, w1v_ref[...], preferred_element_type=jnp.float32)
    h = jnp.maximum(h + b1_ref[...], 0.0)
    y = jnp.sum(h * w2_ref[...], axis=1)
    o_ref[...] = (y[None, :] + b2_ref[...])[None]


_mlp = pl.pallas_call(
    _mlp_body,
    grid=(B // _BLK,),
    in_specs=[
        pl.BlockSpec((_BLK, 128), lambda i: (i, 0)),
        pl.BlockSpec((_BLK, 128), lambda i: (i, 0)),
        pl.BlockSpec((_BLK, 1), lambda i: (i, 0)),
        pl.BlockSpec((_BLK, 1), lambda i: (i, 0)),
        pl.BlockSpec((D, H), lambda i: (0, 0)),
        pl.BlockSpec((D, H), lambda i: (0, 0)),
        pl.BlockSpec((1, H), lambda i: (0, 0)),
        pl.BlockSpec((1, H), lambda i: (0, 0)),
        pl.BlockSpec((1, 1), lambda i: (0, 0)),
    ],
    out_specs=pl.BlockSpec((1, 1, _BLK), lambda i: (i, 0, 0)),
    out_shape=jax.ShapeDtypeStruct((B // _BLK, 1, _BLK), jnp.float32),
)


def kernel(userID, ItemID, user_table, item_table, W1, b1, W2, b2):
    uid = userID.astype(jnp.int32)
    iid = ItemID.astype(jnp.int32)
    upix = ((uid // 1024) * 256 + uid % 256).reshape(B // CHUNK, CHUNK)
    ipix = ((iid // 1024) * 256 + iid % 256).reshape(B // CHUNK, CHUNK)
    ut_c, it_c = _transp(user_table.T, item_table.T)
    u128, v128 = _gather_uv(upix, ipix, ut_c, it_c)
    ru = ((uid // 256) % RPP).astype(jnp.float32).reshape(B, 1)
    rv = ((iid // 256) % RPP).astype(jnp.float32).reshape(B, 1)
    y = _mlp(u128, v128, ru, rv, W1[:D], W1[D:], b1.reshape(1, H),
             W2.reshape(1, H), b2.reshape(1, 1))
    return y.reshape(B)


# speedup vs baseline: 2.7460x; 1.2071x over previous
import functools

import jax
import jax.numpy as jnp
from jax import lax
from jax.experimental import pallas as pl
from jax.experimental.pallas import tpu as pltpu
from jax.experimental.pallas import tpu_sc as plsc

B = 16384
D = 32
H = 64
NROWS = 1000000
RPP = 128 // D
NC = 2
NS = 16
NW = NC * NS
BPW = B // NW
CHUNK = 128
NCHUNK = BPW // CHUNK


_TBLK = 16384
_TGRID = -(-NROWS // _TBLK)


def _transp_body(ut_ref, it_ref, uo_ref, io_ref):
    x = ut_ref[...]
    z = it_ref[...]
    row = jax.lax.broadcasted_iota(jnp.int32, (D, 128), 0)
    col = jax.lax.broadcasted_iota(jnp.int32, (D, 128), 1)
    dn = (((0,), (0,)), ((), ()))
    eas = [(col == row + a * D).astype(jnp.float32) for a in range(RPP)]
    for g in range(_TBLK // 1024):
        u = jnp.zeros((256, 128), jnp.float32)
        v = jnp.zeros((256, 128), jnp.float32)
        for a in range(RPP):
            c0 = g * 1024 + a * 256
            u = u + jax.lax.dot_general(x[:, c0:c0 + 256], eas[a], dn,
                                        preferred_element_type=jnp.float32)
            v = v + jax.lax.dot_general(z[:, c0:c0 + 256], eas[a], dn,
                                        preferred_element_type=jnp.float32)
        uo_ref[pl.ds(g * 256, 256), :] = u
        io_ref[pl.ds(g * 256, 256), :] = v


_transp = pl.pallas_call(
    _transp_body,
    grid=(_TGRID,),
    in_specs=[
        pl.BlockSpec((D, _TBLK), lambda i: (0, i)),
        pl.BlockSpec((D, _TBLK), lambda i: (0, i)),
    ],
    out_specs=[
        pl.BlockSpec((_TBLK // 4, 128), lambda i: (i, 0)),
        pl.BlockSpec((_TBLK // 4, 128), lambda i: (i, 0)),
    ],
    out_shape=[
        jax.ShapeDtypeStruct((_TGRID * (_TBLK // 4), 128), jnp.float32),
        jax.ShapeDtypeStruct((_TGRID * (_TBLK // 4), 128), jnp.float32),
    ],
    compiler_params=pltpu.CompilerParams(fuse_transposed_lhs_in_matmul=True),
)


_mesh = plsc.VectorSubcoreMesh(core_axis_name="c", subcore_axis_name="s")


@functools.partial(
    pl.kernel,
    mesh=_mesh,
    out_type=[
        jax.ShapeDtypeStruct((B, 128), jnp.float32),
        jax.ShapeDtypeStruct((B, 128), jnp.float32),
    ],
    scratch_types=[
        pltpu.VMEM((NCHUNK, CHUNK), jnp.int32),
        pltpu.VMEM((NCHUNK, CHUNK), jnp.int32),
        pltpu.VMEM((BPW // 2, 128), jnp.float32),
        pltpu.VMEM((BPW // 2, 128), jnp.float32),
        pltpu.SemaphoreType.DMA,
        pltpu.SemaphoreType.DMA,
    ],
)
def _gather_uv(uid_hbm, iid_hbm, ut_hbm, it_hbm, u_out, v_out,
               uidx, iidx, urows, vrows, usem, vsem):
    wid = lax.axis_index("s") * NC + lax.axis_index("c")
    base = wid * BPW
    pltpu.sync_copy(uid_hbm.at[pl.ds(wid * NCHUNK, NCHUNK)], uidx)
    pltpu.sync_copy(iid_hbm.at[pl.ds(wid * NCHUNK, NCHUNK)], iidx)
    for half in range(2):
        cps = []
        for j in range(NCHUNK // 2):
            jj = half * (NCHUNK // 2) + j
            cps.append(pltpu.async_copy(
                ut_hbm.at[uidx.at[jj]],
                urows.at[pl.ds(j * CHUNK, CHUNK)], usem))
            cps.append(pltpu.async_copy(
                it_hbm.at[iidx.at[jj]],
                vrows.at[pl.ds(j * CHUNK, CHUNK)], vsem))
        for cp in cps:
            cp.wait()
        pltpu.sync_copy(urows, u_out.at[pl.ds(base + half * (BPW // 2),
                                              BPW // 2)])
        pltpu.sync_copy(vrows, v_out.at[pl.ds(base + half * (BPW // 2),
                                              BPW // 2)])


_BLK = 2048


def _mlp_body(u_ref, v_ref, ru_ref, rv_ref, w1u_ref, w1v_ref, b1_ref,
              w2_ref, b2_ref, o_ref):
    u128 = u_ref[...]
    v128 = v_ref[...]
    ru = ru_ref[...]
    rv = rv_ref[...]
    u = jnp.where(ru == 0., u128[:, 0:D], 0.)
    v = jnp.where(rv == 0., v128[:, 0:D], 0.)
    for k in range(1, RPP):
        u = u + jnp.where(ru == float(k), u128[:, k * D:(k + 1) * D], 0.)
        v = v + jnp.where(rv == float(k), v128[:, k * D:(k + 1) * D], 0.)
    h = jnp.dot(u, w1u_ref[...], preferred_element_type=jnp.float32)
    h = h + jnp.dot(v, w1v_ref[...], preferred_element_type=jnp.float32)
    h = jnp.maximum(h + b1_ref[...], 0.0)
    y = jnp.sum(h * w2_ref[...], axis=1)
    o_ref[...] = (y[None, :] + b2_ref[...])[None]


_mlp = pl.pallas_call(
    _mlp_body,
    grid=(B // _BLK,),
    in_specs=[
        pl.BlockSpec((_BLK, 128), lambda i: (i, 0)),
        pl.BlockSpec((_BLK, 128), lambda i: (i, 0)),
        pl.BlockSpec((_BLK, 1), lambda i: (i, 0)),
        pl.BlockSpec((_BLK, 1), lambda i: (i, 0)),
        pl.BlockSpec((D, H), lambda i: (0, 0)),
        pl.BlockSpec((D, H), lambda i: (0, 0)),
        pl.BlockSpec((1, H), lambda i: (0, 0)),
        pl.BlockSpec((1, H), lambda i: (0, 0)),
        pl.BlockSpec((1, 1), lambda i: (0, 0)),
    ],
    out_specs=pl.BlockSpec((1, 1, _BLK), lambda i: (i, 0, 0)),
    out_shape=jax.ShapeDtypeStruct((B // _BLK, 1, _BLK), jnp.float32),
)


def kernel(userID, ItemID, user_table, item_table, W1, b1, W2, b2):
    uid = userID.astype(jnp.int32)
    iid = ItemID.astype(jnp.int32)
    upix = ((uid // 1024) * 256 + uid % 256).reshape(B // CHUNK, CHUNK)
    ipix = ((iid // 1024) * 256 + iid % 256).reshape(B // CHUNK, CHUNK)
    ut_c, it_c = _transp(user_table.T, item_table.T)
    u128, v128 = _gather_uv(upix, ipix, ut_c, it_c)
    ru = ((uid // 256) % RPP).astype(jnp.float32).reshape(B, 1)
    rv = ((iid // 256) % RPP).astype(jnp.float32).reshape(B, 1)
    y = _mlp(u128, v128, ru, rv, W1[:D], W1[D:], b1.reshape(1, H),
             W2.reshape(1, H), b2.reshape(1, 1))
    return y.reshape(B)


# speedup vs baseline: 2.7737x; 1.0101x over previous
import functools

import jax
import jax.numpy as jnp
from jax import lax
from jax.experimental import pallas as pl
from jax.experimental.pallas import tpu as pltpu
from jax.experimental.pallas import tpu_sc as plsc

B = 16384
D = 32
H = 64
NROWS = 1000000
RPP = 128 // D
NC = 2
NS = 16
NW = NC * NS
BPW = B // NW
CHUNK = 128
NCHUNK = BPW // CHUNK


_TBLK = 32768
_TGRID = -(-NROWS // _TBLK)


def _transp_body(ut_ref, it_ref, uo_ref, io_ref):
    x = ut_ref[...]
    z = it_ref[...]
    row = jax.lax.broadcasted_iota(jnp.int32, (D, 128), 0)
    col = jax.lax.broadcasted_iota(jnp.int32, (D, 128), 1)
    dn = (((0,), (0,)), ((), ()))
    eas = [(col == row + a * D).astype(jnp.float32) for a in range(RPP)]
    for g in range(_TBLK // 1024):
        u = jnp.zeros((256, 128), jnp.float32)
        v = jnp.zeros((256, 128), jnp.float32)
        for a in range(RPP):
            c0 = g * 1024 + a * 256
            u = u + jax.lax.dot_general(x[:, c0:c0 + 256], eas[a], dn,
                                        preferred_element_type=jnp.float32)
            v = v + jax.lax.dot_general(z[:, c0:c0 + 256], eas[a], dn,
                                        preferred_element_type=jnp.float32)
        uo_ref[pl.ds(g * 256, 256), :] = u
        io_ref[pl.ds(g * 256, 256), :] = v


_transp = pl.pallas_call(
    _transp_body,
    grid=(_TGRID,),
    in_specs=[
        pl.BlockSpec((D, _TBLK), lambda i: (0, i)),
        pl.BlockSpec((D, _TBLK), lambda i: (0, i)),
    ],
    out_specs=[
        pl.BlockSpec((_TBLK // 4, 128), lambda i: (i, 0)),
        pl.BlockSpec((_TBLK // 4, 128), lambda i: (i, 0)),
    ],
    out_shape=[
        jax.ShapeDtypeStruct((_TGRID * (_TBLK // 4), 128), jnp.float32),
        jax.ShapeDtypeStruct((_TGRID * (_TBLK // 4), 128), jnp.float32),
    ],
    compiler_params=pltpu.CompilerParams(fuse_transposed_lhs_in_matmul=True),
)


_mesh = plsc.VectorSubcoreMesh(core_axis_name="c", subcore_axis_name="s")


@functools.partial(
    pl.kernel,
    mesh=_mesh,
    out_type=[
        jax.ShapeDtypeStruct((B, 128), jnp.float32),
        jax.ShapeDtypeStruct((B, 128), jnp.float32),
    ],
    scratch_types=[
        pltpu.VMEM((NCHUNK, CHUNK), jnp.int32),
        pltpu.VMEM((NCHUNK, CHUNK), jnp.int32),
        pltpu.VMEM((BPW // 2, 128), jnp.float32),
        pltpu.VMEM((BPW // 2, 128), jnp.float32),
        pltpu.SemaphoreType.DMA,
        pltpu.SemaphoreType.DMA,
    ],
)
def _gather_uv(uid_hbm, iid_hbm, ut_hbm, it_hbm, u_out, v_out,
               uidx, iidx, urows, vrows, usem, vsem):
    wid = lax.axis_index("s") * NC + lax.axis_index("c")
    base = wid * BPW
    pltpu.sync_copy(uid_hbm.at[pl.ds(wid * NCHUNK, NCHUNK)], uidx)
    pltpu.sync_copy(iid_hbm.at[pl.ds(wid * NCHUNK, NCHUNK)], iidx)
    for half in range(2):
        cps = []
        for j in range(NCHUNK // 2):
            jj = half * (NCHUNK // 2) + j
            cps.append(pltpu.async_copy(
                ut_hbm.at[uidx.at[jj]],
                urows.at[pl.ds(j * CHUNK, CHUNK)], usem))
            cps.append(pltpu.async_copy(
                it_hbm.at[iidx.at[jj]],
                vrows.at[pl.ds(j * CHUNK, CHUNK)], vsem))
        for cp in cps:
            cp.wait()
        pltpu.sync_copy(urows, u_out.at[pl.ds(base + half * (BPW // 2),
                                              BPW // 2)])
        pltpu.sync_copy(vrows, v_out.at[pl.ds(base + half * (BPW // 2),
                                              BPW // 2)])


_BLK = 2048


def _mlp_body(u_ref, v_ref, ru_ref, rv_ref, w1u_ref, w1v_ref, b1_ref,
              w2_ref, b2_ref, o_ref):
    u128 = u_ref[...]
    v128 = v_ref[...]
    ru = ru_ref[...]
    rv = rv_ref[...]
    u = jnp.where(ru == 0., u128[:, 0:D], 0.)
    v = jnp.where(rv == 0., v128[:, 0:D], 0.)
    for k in range(1, RPP):
        u = u + jnp.where(ru == float(k), u128[:, k * D:(k + 1) * D], 0.)
        v = v + jnp.where(rv == float(k), v128[:, k * D:(k + 1) * D], 0.)
    h = jnp.dot(u, w1u_ref[...], preferred_element_type=jnp.float32)
    h = h + jnp.dot(v, w1v_ref[...], preferred_element_type=jnp.float32)
    h = jnp.maximum(h + b1_ref[...], 0.0)
    y = jnp.sum(h * w2_ref[...], axis=1)
    o_ref[...] = (y[None, :] + b2_ref[...])[None]


_mlp = pl.pallas_call(
    _mlp_body,
    grid=(B // _BLK,),
    in_specs=[
        pl.BlockSpec((_BLK, 128), lambda i: (i, 0)),
        pl.BlockSpec((_BLK, 128), lambda i: (i, 0)),
        pl.BlockSpec((_BLK, 1), lambda i: (i, 0)),
        pl.BlockSpec((_BLK, 1), lambda i: (i, 0)),
        pl.BlockSpec((D, H), lambda i: (0, 0)),
        pl.BlockSpec((D, H), lambda i: (0, 0)),
        pl.BlockSpec((1, H), lambda i: (0, 0)),
        pl.BlockSpec((1, H), lambda i: (0, 0)),
        pl.BlockSpec((1, 1), lambda i: (0, 0)),
    ],
    out_specs=pl.BlockSpec((1, 1, _BLK), lambda i: (i, 0, 0)),
    out_shape=jax.ShapeDtypeStruct((B // _BLK, 1, _BLK), jnp.float32),
)


def kernel(userID, ItemID, user_table, item_table, W1, b1, W2, b2):
    uid = userID.astype(jnp.int32)
    iid = ItemID.astype(jnp.int32)
    upix = ((uid // 1024) * 256 + uid % 256).reshape(B // CHUNK, CHUNK)
    ipix = ((iid // 1024) * 256 + iid % 256).reshape(B // CHUNK, CHUNK)
    ut_c, it_c = _transp(user_table.T, item_table.T)
    u128, v128 = _gather_uv(upix, ipix, ut_c, it_c)
    ru = ((uid // 256) % RPP).astype(jnp.float32).reshape(B, 1)
    rv = ((iid // 256) % RPP).astype(jnp.float32).reshape(B, 1)
    y = _mlp(u128, v128, ru, rv, W1[:D], W1[D:], b1.reshape(1, H),
             W2.reshape(1, H), b2.reshape(1, 1))
    return y.reshape(B)


# speedup vs baseline: 2.9233x; 1.0539x over previous
import functools

import jax
import jax.numpy as jnp
from jax import lax
from jax.experimental import pallas as pl
from jax.experimental.pallas import tpu as pltpu
from jax.experimental.pallas import tpu_sc as plsc

B = 16384
D = 32
H = 64
NROWS = 1000000
RPP = 128 // D
NC = 2
NS = 16
NW = NC * NS
BPW = B // NW
CHUNK = 128
NCHUNK = BPW // CHUNK


_TBLK = 32768
_TGRID = -(-NROWS // _TBLK)


def _transp_body(ut_ref, it_ref, uo_ref, io_ref):
    x = ut_ref[...]
    z = it_ref[...]
    row = jax.lax.broadcasted_iota(jnp.int32, (D, 128), 0)
    col = jax.lax.broadcasted_iota(jnp.int32, (D, 128), 1)
    dn = (((0,), (0,)), ((), ()))
    eas = [(col == row + a * D).astype(jnp.float32) for a in range(RPP)]
    for g in range(_TBLK // 1024):
        u = jnp.zeros((256, 128), jnp.float32)
        v = jnp.zeros((256, 128), jnp.float32)
        for a in range(RPP):
            c0 = g * 1024 + a * 256
            u = u + jax.lax.dot_general(x[:, c0:c0 + 256], eas[a], dn,
                                        preferred_element_type=jnp.float32)
            v = v + jax.lax.dot_general(z[:, c0:c0 + 256], eas[a], dn,
                                        preferred_element_type=jnp.float32)
        uo_ref[pl.ds(g * 256, 256), :] = u
        io_ref[pl.ds(g * 256, 256), :] = v


_transp = pl.pallas_call(
    _transp_body,
    grid=(_TGRID,),
    in_specs=[
        pl.BlockSpec((D, _TBLK), lambda i: (0, i)),
        pl.BlockSpec((D, _TBLK), lambda i: (0, i)),
    ],
    out_specs=[
        pl.BlockSpec((_TBLK // 4, 128), lambda i: (i, 0)),
        pl.BlockSpec((_TBLK // 4, 128), lambda i: (i, 0)),
    ],
    out_shape=[
        jax.ShapeDtypeStruct((_TGRID * (_TBLK // 4), 128), jnp.float32),
        jax.ShapeDtypeStruct((_TGRID * (_TBLK // 4), 128), jnp.float32),
    ],
    compiler_params=pltpu.CompilerParams(fuse_transposed_lhs_in_matmul=True),
)


_mesh = plsc.VectorSubcoreMesh(core_axis_name="c", subcore_axis_name="s")


@functools.partial(
    pl.kernel,
    mesh=_mesh,
    out_type=[
        jax.ShapeDtypeStruct((B, 128), jnp.float32),
        jax.ShapeDtypeStruct((B, 128), jnp.float32),
    ],
    scratch_types=[
        pltpu.VMEM((NCHUNK, CHUNK), jnp.int32),
        pltpu.VMEM((NCHUNK, CHUNK), jnp.int32),
        pltpu.VMEM((BPW // 2, 128), jnp.float32),
        pltpu.VMEM((BPW // 2, 128), jnp.float32),
        pltpu.SemaphoreType.DMA,
        pltpu.SemaphoreType.DMA,
    ],
)
def _gather_uv(uid_hbm, iid_hbm, ut_hbm, it_hbm, u_out, v_out,
               uidx, iidx, urows, vrows, usem, vsem):
    wid = lax.axis_index("s") * NC + lax.axis_index("c")
    base = wid * BPW
    pltpu.sync_copy(uid_hbm.at[pl.ds(wid * NCHUNK, NCHUNK)], uidx)
    pltpu.sync_copy(iid_hbm.at[pl.ds(wid * NCHUNK, NCHUNK)], iidx)
    for half in range(2):
        cps = []
        for j in range(NCHUNK // 2):
            jj = half * (NCHUNK // 2) + j
            cps.append(pltpu.async_copy(
                ut_hbm.at[uidx.at[jj]],
                urows.at[pl.ds(j * CHUNK, CHUNK)], usem))
            cps.append(pltpu.async_copy(
                it_hbm.at[iidx.at[jj]],
                vrows.at[pl.ds(j * CHUNK, CHUNK)], vsem))
        for cp in cps:
            cp.wait()
        pltpu.sync_copy(urows, u_out.at[pl.ds(base + half * (BPW // 2),
                                              BPW // 2)])
        pltpu.sync_copy(vrows, v_out.at[pl.ds(base + half * (BPW // 2),
                                              BPW // 2)])


_BLK = 2048


def _mlp_body(u_ref, v_ref, ru_ref, rv_ref, cc_ref, w1u_ref, w1v_ref,
              b1_ref, w2_ref, b2_ref, o_ref):
    cc = cc_ref[...]
    um = jnp.where(cc == ru_ref[...], u_ref[...], 0.)
    vm = jnp.where(cc == rv_ref[...], v_ref[...], 0.)
    h = jnp.dot(um, w1u_ref[...], preferred_element_type=jnp.float32)
    h = h + jnp.dot(vm, w1v_ref[...], preferred_element_type=jnp.float32)
    h = jnp.maximum(h + b1_ref[...], 0.0)
    o_ref[...] = (jnp.dot(h, w2_ref[...], preferred_element_type=jnp.float32)
                  + b2_ref[...])


_mlp = pl.pallas_call(
    _mlp_body,
    grid=(B // _BLK,),
    in_specs=[
        pl.BlockSpec((_BLK, 128), lambda i: (i, 0)),
        pl.BlockSpec((_BLK, 128), lambda i: (i, 0)),
        pl.BlockSpec((_BLK, 1), lambda i: (i, 0)),
        pl.BlockSpec((_BLK, 1), lambda i: (i, 0)),
        pl.BlockSpec((1, 128), lambda i: (0, 0)),
        pl.BlockSpec((128, H), lambda i: (0, 0)),
        pl.BlockSpec((128, H), lambda i: (0, 0)),
        pl.BlockSpec((1, H), lambda i: (0, 0)),
        pl.BlockSpec((H, 1), lambda i: (0, 0)),
        pl.BlockSpec((1, 1), lambda i: (0, 0)),
    ],
    out_specs=pl.BlockSpec((_BLK, 1), lambda i: (i, 0)),
    out_shape=jax.ShapeDtypeStruct((B, 1), jnp.float32),
)


def kernel(userID, ItemID, user_table, item_table, W1, b1, W2, b2):
    uid = userID.astype(jnp.int32)
    iid = ItemID.astype(jnp.int32)
    upix = ((uid // 1024) * 256 + uid % 256).reshape(B // CHUNK, CHUNK)
    ipix = ((iid // 1024) * 256 + iid % 256).reshape(B // CHUNK, CHUNK)
    ut_c, it_c = _transp(user_table.T, item_table.T)
    u128, v128 = _gather_uv(upix, ipix, ut_c, it_c)
    ru = ((uid // 256) % RPP).astype(jnp.float32).reshape(B, 1)
    rv = ((iid // 256) % RPP).astype(jnp.float32).reshape(B, 1)
    colchunk = (jnp.arange(128, dtype=jnp.int32) // D).astype(
        jnp.float32).reshape(1, 128)
    w1u_s = jnp.tile(W1[:D], (RPP, 1))
    w1v_s = jnp.tile(W1[D:], (RPP, 1))
    y = _mlp(u128, v128, ru, rv, colchunk, w1u_s, w1v_s, b1.reshape(1, H),
             W2, b2.reshape(1, 1))
    return y.reshape(B)
